# baseline (speedup 1.0000x reference)
import functools

import jax
import jax.numpy as jnp
from jax import lax
from jax.experimental import pallas as pl
from jax.experimental.pallas import tpu as pltpu
from jax.experimental.pallas import tpu_sc as plsc

NUM_SC = 2
NUM_SUBCORES = 16
LANES = 16


def _bf16_top(u):
    r = u + 0x7FFF + (lax.shift_right_logical(u, 16) & 1)
    return lax.shift_right_logical(r, 16)


def _build_dense(vals, cols, n, m):
    k = vals.shape[1]
    nw = NUM_SC * NUM_SUBCORES
    rpt = n // nw
    starts = [c * LANES for c in range(k // LANES)]
    overlap = 0
    if k % LANES:
        starts.append(k - LANES)
        overlap = starts[-2] + LANES - starts[-1] if len(starts) > 1 else 0
    nchunk = len(starts)
    npt = rpt // 2
    grp = 2
    nbuf = 4
    ngroups = npt // grp

    @functools.partial(
        pl.kernel,
        out_type=jax.ShapeDtypeStruct((n // 2, m), jnp.int32),
        mesh=plsc.VectorSubcoreMesh(core_axis_name="c", subcore_axis_name="s"),
        compiler_params=pltpu.CompilerParams(needs_layout_passes=False),
        scratch_types=[
            pltpu.VMEM((rpt, k), jnp.float32),
            pltpu.VMEM((rpt, k), jnp.int32),
            pltpu.VMEM((m,), jnp.float32),
            pltpu.VMEM((m,), jnp.float32),
            pltpu.VMEM((m,), jnp.float32),
            pltpu.VMEM((m,), jnp.float32),
            pltpu.VMEM((grp, m), jnp.int32),
            pltpu.VMEM((grp, m), jnp.int32),
            pltpu.VMEM((grp, m), jnp.int32),
            pltpu.VMEM((grp, m), jnp.int32),
            pltpu.SemaphoreType.DMA,
            pltpu.SemaphoreType.DMA,
            pltpu.SemaphoreType.DMA,
            pltpu.SemaphoreType.DMA,
            pltpu.SemaphoreType.DMA,
            pltpu.SemaphoreType.DMA,
        ],
    )
    def scatter_kernel(vals_hbm, cols_hbm, wd_hbm, vals_v, cols_v,
                       acc0, acc1, acc2, acc3, buf0, buf1, buf2, buf3,
                       sem0, sem1, sem2, sem3, semv, semc):
        wid = lax.axis_index("s") * NUM_SC + lax.axis_index("c")
        base = wid * rpt
        pbase = wid * npt
        cp_v = pltpu.async_copy(vals_hbm.at[pl.ds(base, rpt)], vals_v, semv)
        cp_c = pltpu.async_copy(cols_hbm.at[pl.ds(base, rpt)], cols_v, semc)

        zero16f = jnp.zeros((LANES,), jnp.float32)
        zero16i = jnp.zeros((LANES,), jnp.int32)
        lane_id = jax.lax.iota(jnp.int32, LANES)
        chunk_masks = [lane_id >= (overlap if c == nchunk - 1 else 0)
                       for c in range(nchunk)]
        bufs = (buf0, buf1, buf2, buf3)
        sems = (sem0, sem1, sem2, sem3)
        accsets = ((acc0, acc1), (acc2, acc3))

        def zinit(i, carry):
            for gg in range(grp):
                for bb in bufs:
                    bb[gg, pl.ds(i * LANES, LANES)] = zero16i
            for aset in accsets:
                for a in aset:
                    a[pl.ds(i * LANES, LANES)] = zero16f
            return carry

        lax.fori_loop(0, m // LANES, zinit, 0)
        cp_v.wait()
        cp_c.wait()

        def do_pair_duo(buf, p0):
            us = range(grp)
            row_ids = [jnp.full((LANES,), u, jnp.int32) for u in us]
            idxs = [[cols_v[2 * (p0 + u) + h, pl.ds(s, LANES)]
                     for h in (0, 1) for s in starts] for u in us]
            for c in range(nchunk):
                for u in us:
                    for h in (0, 1):
                        r = 2 * (p0 + u) + h
                        v = vals_v[r, pl.ds(starts[c], LANES)]
                        plsc.addupdate_scatter(accsets[u][h],
                                               [idxs[u][h * nchunk + c]], v,
                                               mask=chunk_masks[c])
            for c in range(2 * nchunk):
                for u in us:
                    idx = idxs[u][c]
                    a = plsc.load_gather(accsets[u][0], [idx])
                    b = plsc.load_gather(accsets[u][1], [idx])
                    ta = _bf16_top(plsc.bitcast(a, jnp.int32))
                    tb = _bf16_top(plsc.bitcast(b, jnp.int32))
                    word = lax.shift_left(tb, 16) | ta
                    plsc.store_scatter(buf, [row_ids[u], idx], word)
            for c in range(nchunk):
                for u in us:
                    for h in (0, 1):
                        plsc.store_scatter(accsets[u][h],
                                           [idxs[u][h * nchunk + c]], zero16f)

        def round_body(t, carry):
            for bsel in range(nbuf):
                g = t * nbuf + bsel
                buf = bufs[bsel]
                sem = sems[bsel]

                @pl.when(t > 0)
                def _():
                    pltpu.make_async_copy(
                        buf, wd_hbm.at[pl.ds(pbase, grp)], sem).wait()
                    for u in range(grp):
                        row_id = jnp.full((LANES,), u, jnp.int32)
                        old_p = (g - nbuf) * grp + u
                        for h in (0, 1):
                            for c in range(nchunk):
                                idx = cols_v[2 * old_p + h,
                                             pl.ds(starts[c], LANES)]
                                plsc.store_scatter(buf, [row_id, idx],
                                                   zero16i)

                do_pair_duo(buf, g * grp)

                pltpu.async_copy(buf, wd_hbm.at[pl.ds(pbase + g * grp, grp)],
                                 sem)
            return carry

        lax.fori_loop(0, ngroups // nbuf, round_body, 0)
        for bsel in range(nbuf):
            pltpu.make_async_copy(bufs[bsel], wd_hbm.at[pl.ds(pbase, grp)],
                                  sems[bsel]).wait()

    return scatter_kernel(vals, cols)


def _matmul_body(x_ref, wd_ref, bias_ref, out_ref):
    wb = pltpu.bitcast(wd_ref[...], jnp.bfloat16)
    xb = x_ref[...].astype(jnp.bfloat16)
    acc = lax.dot_general(xb, wb, (((1,), (1,)), ((), ())),
                          preferred_element_type=jnp.float32)
    nb = out_ref.shape[1]
    j = pl.program_id(0)
    out_ref[...] = acc + bias_ref[pl.ds(j * nb, nb)][None, :]


def _matmul(xb, wd32, bias, nb=2048):
    b, m = xb.shape
    n = wd32.shape[0] * 2
    return pl.pallas_call(
        _matmul_body,
        grid=(n // nb,),
        in_specs=[
            pl.BlockSpec((b, m), lambda i: (0, 0)),
            pl.BlockSpec((nb // 2, m), lambda i: (i, 0)),
            pl.BlockSpec((n,), lambda i: (0,)),
        ],
        out_specs=pl.BlockSpec((b, nb), lambda i: (0, i)),
        out_shape=jax.ShapeDtypeStruct((b, n), jnp.float32),
        compiler_params=pltpu.CompilerParams(
            dimension_semantics=("arbitrary",)),
    )(xb, wd32, bias)


def kernel(input, W_val, W_cols, bias):
    b, m = input.shape
    n, k = W_val.shape
    wd32 = _build_dense(W_val, W_cols.astype(jnp.int32), n, m)
    return _matmul(input, wd32, bias)


# speedup vs baseline: 1.0014x; 1.0014x over previous
import functools

import jax
import jax.numpy as jnp
from jax import lax
from jax.experimental import pallas as pl
from jax.experimental.pallas import tpu as pltpu
from jax.experimental.pallas import tpu_sc as plsc

NUM_SC = 2
NUM_SUBCORES = 16
LANES = 16


def _bf16_top(u):
    r = u + 0x7FFF + (lax.shift_right_logical(u, 16) & 1)
    return lax.shift_right_logical(r, 16)


def _build_dense(vals, cols, n, m):
    k = vals.shape[1]
    nw = NUM_SC * NUM_SUBCORES
    rpt = n // nw
    starts = [c * LANES for c in range(k // LANES)]
    overlap = 0
    if k % LANES:
        starts.append(k - LANES)
        overlap = starts[-2] + LANES - starts[-1] if len(starts) > 1 else 0
    nchunk = len(starts)
    npt = rpt // 2
    grp = 2
    nbuf = 4
    ngroups = npt // grp

    @functools.partial(
        pl.kernel,
        out_type=jax.ShapeDtypeStruct((n // 2, m), jnp.int32),
        mesh=plsc.VectorSubcoreMesh(core_axis_name="c", subcore_axis_name="s"),
        compiler_params=pltpu.CompilerParams(needs_layout_passes=False),
        scratch_types=[
            pltpu.VMEM((rpt, k), jnp.float32),
            pltpu.VMEM((rpt, k), jnp.int32),
            pltpu.VMEM((m,), jnp.float32),
            pltpu.VMEM((m,), jnp.float32),
            pltpu.VMEM((m,), jnp.float32),
            pltpu.VMEM((m,), jnp.float32),
            pltpu.VMEM((grp, m), jnp.int32),
            pltpu.VMEM((grp, m), jnp.int32),
            pltpu.VMEM((grp, m), jnp.int32),
            pltpu.VMEM((grp, m), jnp.int32),
            pltpu.SemaphoreType.DMA,
            pltpu.SemaphoreType.DMA,
            pltpu.SemaphoreType.DMA,
            pltpu.SemaphoreType.DMA,
            pltpu.SemaphoreType.DMA,
            pltpu.SemaphoreType.DMA,
        ],
    )
    def scatter_kernel(vals_hbm, cols_hbm, wd_hbm, vals_v, cols_v,
                       acc0, acc1, acc2, acc3, buf0, buf1, buf2, buf3,
                       sem0, sem1, sem2, sem3, semv, semc):
        wid = lax.axis_index("s") * NUM_SC + lax.axis_index("c")
        base = wid * rpt
        pbase = wid * npt
        cp_v = pltpu.async_copy(vals_hbm.at[pl.ds(base, rpt)], vals_v, semv)
        cp_c = pltpu.async_copy(cols_hbm.at[pl.ds(base, rpt)], cols_v, semc)

        zero16f = jnp.zeros((LANES,), jnp.float32)
        zero16i = jnp.zeros((LANES,), jnp.int32)
        lane_id = jax.lax.iota(jnp.int32, LANES)
        chunk_masks = [lane_id >= (overlap if c == nchunk - 1 else 0)
                       for c in range(nchunk)]
        bufs = (buf0, buf1, buf2, buf3)
        sems = (sem0, sem1, sem2, sem3)
        accsets = ((acc0, acc1), (acc2, acc3))

        def zinit(i, carry):
            for gg in range(grp):
                for bb in bufs:
                    bb[gg, pl.ds(i * LANES, LANES)] = zero16i
            for aset in accsets:
                for a in aset:
                    a[pl.ds(i * LANES, LANES)] = zero16f
            return carry

        lax.fori_loop(0, m // LANES, zinit, 0)
        cp_v.wait()
        cp_c.wait()

        def do_pair_duo(buf, p0):
            us = range(grp)
            row_ids = [jnp.full((LANES,), u, jnp.int32) for u in us]
            idxs = [[cols_v[2 * (p0 + u) + h, pl.ds(s, LANES)]
                     for h in (0, 1) for s in starts] for u in us]
            for c in range(nchunk):
                for u in us:
                    for h in (0, 1):
                        r = 2 * (p0 + u) + h
                        v = vals_v[r, pl.ds(starts[c], LANES)]
                        plsc.addupdate_scatter(accsets[u][h],
                                               [idxs[u][h * nchunk + c]], v,
                                               mask=chunk_masks[c])
            for c in range(2 * nchunk):
                for u in us:
                    idx = idxs[u][c]
                    a = plsc.load_gather(accsets[u][0], [idx])
                    b = plsc.load_gather(accsets[u][1], [idx])
                    ta = _bf16_top(plsc.bitcast(a, jnp.int32))
                    tb = _bf16_top(plsc.bitcast(b, jnp.int32))
                    word = lax.shift_left(tb, 16) | ta
                    plsc.store_scatter(buf, [row_ids[u], idx], word)
            for c in range(nchunk):
                for u in us:
                    for h in (0, 1):
                        plsc.store_scatter(accsets[u][h],
                                           [idxs[u][h * nchunk + c]], zero16f)

        def round_body(t, carry):
            for bsel in range(nbuf):
                g = t * nbuf + bsel
                buf = bufs[bsel]
                sem = sems[bsel]

                @pl.when(t > 0)
                def _():
                    pltpu.make_async_copy(
                        buf, wd_hbm.at[pl.ds(pbase, grp)], sem).wait()
                    for u in range(grp):
                        row_id = jnp.full((LANES,), u, jnp.int32)
                        old_p = (g - nbuf) * grp + u
                        for h in (0, 1):
                            for c in range(nchunk):
                                idx = cols_v[2 * old_p + h,
                                             pl.ds(starts[c], LANES)]
                                plsc.store_scatter(buf, [row_id, idx],
                                                   zero16i)

                do_pair_duo(buf, g * grp)

                pltpu.async_copy(buf, wd_hbm.at[pl.ds(pbase + g * grp, grp)],
                                 sem)
            return carry

        lax.fori_loop(0, ngroups // nbuf, round_body, 0)
        for bsel in range(nbuf):
            pltpu.make_async_copy(bufs[bsel], wd_hbm.at[pl.ds(pbase, grp)],
                                  sems[bsel]).wait()

    return scatter_kernel(vals, cols)


def _matmul_body(x_ref, wd_ref, bias_ref, out_ref):
    wb = pltpu.bitcast(wd_ref[...], jnp.bfloat16)
    acc = lax.dot_general(x_ref[...], wb, (((1,), (1,)), ((), ())),
                          preferred_element_type=jnp.float32)
    nb = out_ref.shape[1]
    j = pl.program_id(0)
    out_ref[...] = acc + bias_ref[pl.ds(j * nb, nb)][None, :]


def _matmul(xb, wd32, bias, nb=2048):
    b, m = xb.shape
    n = wd32.shape[0] * 2
    return pl.pallas_call(
        _matmul_body,
        grid=(n // nb,),
        in_specs=[
            pl.BlockSpec((b, m), lambda i: (0, 0)),
            pl.BlockSpec((nb // 2, m), lambda i: (i, 0)),
            pl.BlockSpec((n,), lambda i: (0,)),
        ],
        out_specs=pl.BlockSpec((b, nb), lambda i: (0, i)),
        out_shape=jax.ShapeDtypeStruct((b, n), jnp.float32),
        compiler_params=pltpu.CompilerParams(
            dimension_semantics=("arbitrary",)),
    )(xb, wd32, bias)


def kernel(input, W_val, W_cols, bias):
    b, m = input.shape
    n, k = W_val.shape
    wd32 = _build_dense(W_val, W_cols.astype(jnp.int32), n, m)
    xb = input.astype(jnp.bfloat16)
    return _matmul(xb, wd32, bias)


# speedup vs baseline: 1.0182x; 1.0168x over previous
import functools

import jax
import jax.numpy as jnp
from jax import lax
from jax.experimental import pallas as pl
from jax.experimental.pallas import tpu as pltpu
from jax.experimental.pallas import tpu_sc as plsc

NUM_SC = 2
NUM_SUBCORES = 16
LANES = 16


def _bf16_top(u):
    r = u + 0x7FFF + (lax.shift_right_logical(u, 16) & 1)
    return lax.shift_right_logical(r, 16)


def _build_dense(vals, cols, n, m):
    kp = vals.shape[1]
    nw = NUM_SC * NUM_SUBCORES
    rpt = n // nw
    starts = [c * LANES for c in range(kp // LANES)]
    nchunk = len(starts)
    npt = rpt // 2
    grp = 2
    nbuf = 4
    ngroups = npt // grp

    @functools.partial(
        pl.kernel,
        out_type=jax.ShapeDtypeStruct((n // 2, m), jnp.int32),
        mesh=plsc.VectorSubcoreMesh(core_axis_name="c", subcore_axis_name="s"),
        compiler_params=pltpu.CompilerParams(needs_layout_passes=False),
        scratch_types=[
            pltpu.VMEM((rpt, kp), jnp.float32),
            pltpu.VMEM((rpt, kp), jnp.int32),
            pltpu.VMEM((m,), jnp.float32),
            pltpu.VMEM((m,), jnp.float32),
            pltpu.VMEM((m,), jnp.float32),
            pltpu.VMEM((m,), jnp.float32),
            pltpu.VMEM((grp, m), jnp.int32),
            pltpu.VMEM((grp, m), jnp.int32),
            pltpu.VMEM((grp, m), jnp.int32),
            pltpu.VMEM((grp, m), jnp.int32),
            pltpu.SemaphoreType.DMA,
            pltpu.SemaphoreType.DMA,
            pltpu.SemaphoreType.DMA,
            pltpu.SemaphoreType.DMA,
            pltpu.SemaphoreType.DMA,
            pltpu.SemaphoreType.DMA,
        ],
    )
    def scatter_kernel(vals_hbm, cols_hbm, wd_hbm, vals_v, cols_v,
                       acc0, acc1, acc2, acc3, buf0, buf1, buf2, buf3,
                       sem0, sem1, sem2, sem3, semv, semc):
        wid = lax.axis_index("s") * NUM_SC + lax.axis_index("c")
        base = wid * rpt
        pbase = wid * npt
        cp_v = pltpu.async_copy(vals_hbm.at[pl.ds(base, rpt)], vals_v, semv)
        cp_c = pltpu.async_copy(cols_hbm.at[pl.ds(base, rpt)], cols_v, semc)

        zero16f = jnp.zeros((LANES,), jnp.float32)
        zero16i = jnp.zeros((LANES,), jnp.int32)
        bufs = (buf0, buf1, buf2, buf3)
        sems = (sem0, sem1, sem2, sem3)
        accsets = ((acc0, acc1), (acc2, acc3))

        def zinit(i, carry):
            for gg in range(grp):
                for bb in bufs:
                    bb[gg, pl.ds(i * LANES, LANES)] = zero16i
            for aset in accsets:
                for a in aset:
                    a[pl.ds(i * LANES, LANES)] = zero16f
            return carry

        lax.fori_loop(0, m // LANES, zinit, 0)
        cp_v.wait()
        cp_c.wait()

        def do_pair_duo(buf, p0):
            us = range(grp)
            row_ids = [jnp.full((LANES,), u, jnp.int32) for u in us]
            idxs = [[cols_v[2 * (p0 + u) + h, pl.ds(s, LANES)]
                     for h in (0, 1) for s in starts] for u in us]
            for c in range(nchunk):
                for u in us:
                    for h in (0, 1):
                        r = 2 * (p0 + u) + h
                        v = vals_v[r, pl.ds(starts[c], LANES)]
                        plsc.addupdate_scatter(accsets[u][h],
                                               [idxs[u][h * nchunk + c]], v)
            for c in range(2 * nchunk):
                for u in us:
                    idx = idxs[u][c]
                    a = plsc.load_gather(accsets[u][0], [idx])
                    b = plsc.load_gather(accsets[u][1], [idx])
                    ta = _bf16_top(plsc.bitcast(a, jnp.int32))
                    tb = _bf16_top(plsc.bitcast(b, jnp.int32))
                    word = lax.shift_left(tb, 16) | ta
                    plsc.store_scatter(buf, [row_ids[u], idx], word)
            for c in range(nchunk):
                for u in us:
                    for h in (0, 1):
                        plsc.store_scatter(accsets[u][h],
                                           [idxs[u][h * nchunk + c]], zero16f)

        def round_body(t, carry):
            for bsel in range(nbuf):
                g = t * nbuf + bsel
                buf = bufs[bsel]
                sem = sems[bsel]

                @pl.when(t > 0)
                def _():
                    pltpu.make_async_copy(
                        buf, wd_hbm.at[pl.ds(pbase, grp)], sem).wait()
                    for u in range(grp):
                        row_id = jnp.full((LANES,), u, jnp.int32)
                        old_p = (g - nbuf) * grp + u
                        for h in (0, 1):
                            for c in range(nchunk):
                                idx = cols_v[2 * old_p + h,
                                             pl.ds(starts[c], LANES)]
                                plsc.store_scatter(buf, [row_id, idx],
                                                   zero16i)

                do_pair_duo(buf, g * grp)

                pltpu.async_copy(buf, wd_hbm.at[pl.ds(pbase + g * grp, grp)],
                                 sem)
            return carry

        lax.fori_loop(0, ngroups // nbuf, round_body, 0)
        for bsel in range(nbuf):
            pltpu.make_async_copy(bufs[bsel], wd_hbm.at[pl.ds(pbase, grp)],
                                  sems[bsel]).wait()

    return scatter_kernel(vals, cols)


def _matmul_body(x_ref, wd_ref, bias_ref, out_ref):
    wb = pltpu.bitcast(wd_ref[...], jnp.bfloat16)
    acc = lax.dot_general(x_ref[...], wb, (((1,), (1,)), ((), ())),
                          preferred_element_type=jnp.float32)
    nb = out_ref.shape[1]
    j = pl.program_id(0)
    out_ref[...] = acc + bias_ref[pl.ds(j * nb, nb)][None, :]


def _matmul(xb, wd32, bias, nb=2048):
    b, m = xb.shape
    n = wd32.shape[0] * 2
    return pl.pallas_call(
        _matmul_body,
        grid=(n // nb,),
        in_specs=[
            pl.BlockSpec((b, m), lambda i: (0, 0)),
            pl.BlockSpec((nb // 2, m), lambda i: (i, 0)),
            pl.BlockSpec((n,), lambda i: (0,)),
        ],
        out_specs=pl.BlockSpec((b, nb), lambda i: (0, i)),
        out_shape=jax.ShapeDtypeStruct((b, n), jnp.float32),
        compiler_params=pltpu.CompilerParams(
            dimension_semantics=("arbitrary",)),
    )(xb, wd32, bias)


def kernel(input, W_val, W_cols, bias):
    b, m = input.shape
    n, k = W_val.shape
    kp = ((k + LANES - 1) // LANES) * LANES
    vals = jnp.pad(W_val, ((0, 0), (0, kp - k)))
    cols = jnp.pad(W_cols.astype(jnp.int32), ((0, 0), (0, kp - k)))

    wd32 = _build_dense(vals, cols, n, m)
    xb = input.astype(jnp.bfloat16)
    return _matmul(xb, wd32, bias)
